# baseline (speedup 1.0000x reference)
import functools

import jax
import jax.numpy as jnp
import numpy as np
from jax import lax
from jax.experimental import pallas as pl
from jax.experimental.pallas import tpu as pltpu
from jax.experimental.pallas import tpu_sc as plsc

_LANES = 16
_L2 = 32
_NC = 2
_NS = 16
_NW = _NC * _NS
_K = 256


def _matmul_block(x_ref, w_ref, b_ref, out_ref):
    out_ref[...] = (lax.dot_general(
        x_ref[...], w_ref[...], (((1,), (1,)), ((), ())),
        preferred_element_type=jnp.float32) + b_ref[...]
    ).astype(jnp.bfloat16)


def _add_block(p_ref, out_ref):
    out_ref[...] = (p_ref[0].astype(jnp.float32)
                    + p_ref[1].astype(jnp.float32))


def _feature_perm(d):
    perm = np.empty((d,), dtype=np.int32)
    for j in range(d // _L2):
        for t in range(_LANES):
            perm[_L2 * j + 2 * t] = _L2 * j + t
            perm[_L2 * j + 2 * t + 1] = _L2 * j + _LANES + t
    return perm


def _make_scatter(n_pad, d, ep):
    per_tile = ep // _NW
    blocks = per_tile // _K
    nsub = d // _L2
    k8 = _K * nsub
    sub_rows = n_pad * nsub
    sub_per_tile = sub_rows // _NS
    mesh = plsc.VectorSubcoreMesh(core_axis_name="c", subcore_axis_name="s")

    @functools.partial(
        pl.kernel,
        out_type=jax.ShapeDtypeStruct((_NC, n_pad, d), jnp.bfloat16),
        mesh=mesh,
        compiler_params=pltpu.CompilerParams(use_tc_tiling_on_sc=False,
                                             needs_layout_passes=False),
        scratch_types=[
            pltpu.VMEM((_K,), jnp.int32),
            pltpu.VMEM((k8,), jnp.int32),
            pltpu.VMEM((_K * _LANES,), jnp.float32),
            pltpu.VMEM((_K, d), jnp.bfloat16),
            pltpu.VMEM((k8, _L2), jnp.bfloat16),
            pltpu.VMEM_SHARED((sub_rows, _L2), jnp.bfloat16),
            pltpu.SemaphoreType.DMA,
        ],
    )
    def scatter(support_hbm, src_hbm, dst4_hbm, w_hbm, out_hbm,
                src_v, dst4_v, w_v, rows_v, rows8_v, acc, sem):
        cid = lax.axis_index("c")
        sid = lax.axis_index("s")
        ebase = (cid * _NS + sid) * per_tile
        rbase = sid * sub_per_tile

        def zero_row(r, carry):
            rows8_v[r, :] = jnp.zeros((_L2,), jnp.bfloat16)
            return carry
        lax.fori_loop(0, k8, zero_row, 0)
        for i in range(sub_per_tile // k8):
            pltpu.sync_copy(rows8_v.at[pl.ds(0, k8)],
                            acc.at[pl.ds(rbase + i * k8, k8)])
        plsc.subcore_barrier()

        def block_body(blk, carry):
            eb = pl.multiple_of(ebase + blk * _K, _K)
            pltpu.sync_copy(src_hbm.at[pl.ds(eb, _K)], src_v)
            pltpu.sync_copy(
                dst4_hbm.at[pl.ds(pl.multiple_of(eb * nsub, k8), k8)], dst4_v)
            pltpu.sync_copy(
                w_hbm.at[pl.ds(pl.multiple_of(eb * _LANES, _K * _LANES),
                               _K * _LANES)], w_v)
            pltpu.async_copy(support_hbm.at[src_v], rows_v, sem).wait()

            def edge_body(e, c2):
                wv = w_v[pl.ds(pl.multiple_of(e * _LANES, _LANES), _LANES)]
                e4 = e * nsub
                for j in range(nsub):
                    v = rows_v[e, pl.ds(j * _L2, _L2)]
                    a, b2 = plsc.unpack(v, format=plsc.PackFormat.INTERLEAVED)
                    rows8_v[e4 + j, :] = plsc.pack(
                        a * wv, b2 * wv, format=plsc.PackFormat.INTERLEAVED)
                return c2
            lax.fori_loop(0, _K, edge_body, 0)

            pltpu.sync_copy(rows8_v, acc.at[dst4_v], add=True)
            return carry
        lax.fori_loop(0, blocks, block_body, 0)

        plsc.subcore_barrier()
        rowbase = sid * (n_pad // _NS)
        rows_per_chunk = k8 // nsub
        for i in range(sub_per_tile // k8):
            pltpu.sync_copy(acc.at[pl.ds(rbase + i * k8, k8)], rows8_v)

            def repack_row(r, carry):
                r4 = r * nsub
                for j in range(nsub):
                    rows_v[r, pl.ds(j * _L2, _L2)] = rows8_v[r4 + j, :]
                return carry
            lax.fori_loop(0, rows_per_chunk, repack_row, 0)
            pltpu.sync_copy(
                rows_v.at[pl.ds(0, rows_per_chunk)],
                out_hbm.at[cid, pl.ds(rowbase + i * rows_per_chunk,
                                      rows_per_chunk)])

    return scatter


def kernel(input, edge_index, edge_weight, W, b):
    n, d_in = input.shape
    d_out = W.shape[0]
    e = edge_weight.shape[0]
    assert d_in % _LANES == 0 and d_out % _L2 == 0
    n_pad = -(-n // (_NS * _K)) * (_NS * _K)

    perm = _feature_perm(d_out)
    inv = np.argsort(perm)
    W_p = W[jnp.asarray(perm)]
    b_p = b[jnp.asarray(perm)]

    rb = 1000
    grid = (n // rb,)
    support = pl.pallas_call(
        _matmul_block,
        grid=grid,
        in_specs=[pl.BlockSpec((rb, d_in), lambda i: (i, 0)),
                  pl.BlockSpec((d_out, d_in), lambda i: (0, 0)),
                  pl.BlockSpec((1, d_out), lambda i: (0, 0))],
        out_specs=pl.BlockSpec((rb, d_out), lambda i: (i, 0)),
        out_shape=jax.ShapeDtypeStruct((n, d_out), jnp.bfloat16),
    )(input, W_p, b_p.reshape(1, d_out))

    chunk = _NW * _K
    ep = ((e + chunk - 1) // chunk) * chunk
    pad = ep - e
    src = jnp.concatenate(
        [edge_index[1].astype(jnp.int32), jnp.zeros((pad,), jnp.int32)])
    dst = jnp.concatenate(
        [edge_index[0].astype(jnp.int32), jnp.zeros((pad,), jnp.int32)])
    w = jnp.concatenate(
        [edge_weight.astype(jnp.float32), jnp.zeros((pad,), jnp.float32)])
    w = jnp.broadcast_to(w[:, None], (ep, _LANES)).reshape(ep * _LANES)
    nsub = d_out // _L2
    dst4 = (dst[:, None] * nsub + jnp.arange(nsub, dtype=jnp.int32)
            ).reshape(ep * nsub)

    partials = _make_scatter(n_pad, d_out, ep)(support, src, dst4, w)

    out_p = pl.pallas_call(
        _add_block,
        grid=grid,
        in_specs=[pl.BlockSpec((_NC, rb, d_out), lambda i: (0, i, 0))],
        out_specs=pl.BlockSpec((rb, d_out), lambda i: (i, 0)),
        out_shape=jax.ShapeDtypeStruct((n, d_out), jnp.float32),
    )(partials)
    return out_p[:, jnp.asarray(inv)]


# speedup vs baseline: 1.2694x; 1.2694x over previous
import functools

import jax
import jax.numpy as jnp
import numpy as np
from jax import lax
from jax.experimental import pallas as pl
from jax.experimental.pallas import tpu as pltpu
from jax.experimental.pallas import tpu_sc as plsc

_LANES = 16
_L2 = 32
_NC = 2
_NS = 16
_NW = _NC * _NS
_K = 128


def _matmul_block(x_ref, w_ref, b_ref, out_ref):
    out_ref[...] = (lax.dot_general(
        x_ref[...], w_ref[...], (((1,), (1,)), ((), ())),
        preferred_element_type=jnp.float32) + b_ref[...]
    ).astype(jnp.bfloat16)


def _add_block(p_ref, out_ref):
    out_ref[...] = (p_ref[0].astype(jnp.float32)
                    + p_ref[1].astype(jnp.float32))


def _feature_perm(d):
    perm = np.empty((d,), dtype=np.int32)
    for j in range(d // _L2):
        for t in range(_LANES):
            perm[_L2 * j + 2 * t] = _L2 * j + t
            perm[_L2 * j + 2 * t + 1] = _L2 * j + _LANES + t
    return perm


def _make_scatter(n_pad, d, ep):
    per_tile = ep // _NW
    blocks = per_tile // _K
    nsub = d // _L2
    k8 = _K * nsub
    sub_rows = n_pad * nsub
    sub_per_tile = sub_rows // _NS
    mesh = plsc.VectorSubcoreMesh(core_axis_name="c", subcore_axis_name="s")

    @functools.partial(
        pl.kernel,
        out_type=jax.ShapeDtypeStruct((_NC, n_pad, d), jnp.bfloat16),
        mesh=mesh,
        compiler_params=pltpu.CompilerParams(use_tc_tiling_on_sc=False,
                                             needs_layout_passes=False),
        scratch_types=[
            pltpu.VMEM((_K,), jnp.int32),
            pltpu.VMEM((k8,), jnp.int32),
            pltpu.VMEM((_K * _LANES,), jnp.float32),
            pltpu.VMEM((_K, d), jnp.bfloat16),
            pltpu.VMEM((k8, _L2), jnp.bfloat16),
            pltpu.VMEM_SHARED((sub_rows, _L2), jnp.bfloat16),
            pltpu.SemaphoreType.DMA,
        ],
    )
    def scatter(support_hbm, src_hbm, dst4_hbm, w_hbm, out_hbm,
                src_v, dst4_v, w_v, rows_v, rows8_v, acc, sem):
        cid = lax.axis_index("c")
        sid = lax.axis_index("s")
        ebase = (cid * _NS + sid) * per_tile
        rbase = sid * sub_per_tile

        def zero_row(r, carry):
            rows8_v[r, :] = jnp.zeros((_L2,), jnp.bfloat16)
            return carry
        lax.fori_loop(0, k8, zero_row, 0)
        for i in range(sub_per_tile // k8):
            pltpu.sync_copy(rows8_v.at[pl.ds(0, k8)],
                            acc.at[pl.ds(rbase + i * k8, k8)])
        plsc.subcore_barrier()

        def block_body(blk, carry):
            eb = pl.multiple_of(ebase + blk * _K, _K)
            pltpu.sync_copy(src_hbm.at[pl.ds(eb, _K)], src_v)
            pltpu.sync_copy(
                dst4_hbm.at[pl.ds(pl.multiple_of(eb * nsub, k8), k8)], dst4_v)
            pltpu.sync_copy(
                w_hbm.at[pl.ds(pl.multiple_of(eb * _LANES, _K * _LANES),
                               _K * _LANES)], w_v)
            pltpu.async_copy(support_hbm.at[src_v], rows_v, sem).wait()

            def edge_body(e, c2):
                wv = w_v[pl.ds(pl.multiple_of(e * _LANES, _LANES), _LANES)]
                e4 = e * nsub
                for j in range(nsub):
                    v = rows_v[e, pl.ds(j * _L2, _L2)]
                    a, b2 = plsc.unpack(v, format=plsc.PackFormat.INTERLEAVED)
                    rows8_v[e4 + j, :] = plsc.pack(
                        a * wv, b2 * wv, format=plsc.PackFormat.INTERLEAVED)
                return c2
            lax.fori_loop(0, _K, edge_body, 0)

            pltpu.sync_copy(rows8_v, acc.at[dst4_v], add=True)
            return carry
        lax.fori_loop(0, blocks, block_body, 0)

        plsc.subcore_barrier()
        rowbase = sid * (n_pad // _NS)
        rows_per_chunk = k8 // nsub
        for i in range(sub_per_tile // k8):
            pltpu.sync_copy(acc.at[pl.ds(rbase + i * k8, k8)], rows8_v)

            def repack_row(r, carry):
                r4 = r * nsub
                for j in range(nsub):
                    rows_v[r, pl.ds(j * _L2, _L2)] = rows8_v[r4 + j, :]
                return carry
            lax.fori_loop(0, rows_per_chunk, repack_row, 0)
            pltpu.sync_copy(
                rows_v.at[pl.ds(0, rows_per_chunk)],
                out_hbm.at[cid, pl.ds(rowbase + i * rows_per_chunk,
                                      rows_per_chunk)])

    return scatter


def kernel(input, edge_index, edge_weight, W, b):
    n, d_in = input.shape
    d_out = W.shape[0]
    e = edge_weight.shape[0]
    assert d_in % _LANES == 0 and d_out % _L2 == 0
    n_pad = -(-n // (_NS * _K)) * (_NS * _K)

    perm = _feature_perm(d_out)
    inv = np.argsort(perm)
    W_p = W[jnp.asarray(perm)]
    b_p = b[jnp.asarray(perm)]

    rb = 1000
    grid = (n // rb,)
    support = pl.pallas_call(
        _matmul_block,
        grid=grid,
        in_specs=[pl.BlockSpec((rb, d_in), lambda i: (i, 0)),
                  pl.BlockSpec((d_out, d_in), lambda i: (0, 0)),
                  pl.BlockSpec((1, d_out), lambda i: (0, 0))],
        out_specs=pl.BlockSpec((rb, d_out), lambda i: (i, 0)),
        out_shape=jax.ShapeDtypeStruct((n, d_out), jnp.bfloat16),
    )(input, W_p, b_p.reshape(1, d_out))

    chunk = _NW * _K
    ep = ((e + chunk - 1) // chunk) * chunk
    pad = ep - e
    src = jnp.concatenate(
        [edge_index[1].astype(jnp.int32), jnp.zeros((pad,), jnp.int32)])
    dst = jnp.concatenate(
        [edge_index[0].astype(jnp.int32), jnp.zeros((pad,), jnp.int32)])
    w = jnp.concatenate(
        [edge_weight.astype(jnp.float32), jnp.zeros((pad,), jnp.float32)])
    w = jnp.broadcast_to(w[:, None], (ep, _LANES)).reshape(ep * _LANES)
    nsub = d_out // _L2
    dst4 = (dst[:, None] * nsub + jnp.arange(nsub, dtype=jnp.int32)
            ).reshape(ep * nsub)

    partials = _make_scatter(n_pad, d_out, ep)(support, src, dst4, w)

    out_p = pl.pallas_call(
        _add_block,
        grid=grid,
        in_specs=[pl.BlockSpec((_NC, rb, d_out), lambda i: (0, i, 0))],
        out_specs=pl.BlockSpec((rb, d_out), lambda i: (i, 0)),
        out_shape=jax.ShapeDtypeStruct((n, d_out), jnp.float32),
    )(partials)
    return out_p[:, jnp.asarray(inv)]
